# baseline (speedup 1.0000x reference)
import functools

import jax
import jax.numpy as jnp
from jax.experimental import pallas as pl

_HIST = 50
_WIDTH = 74
_HIST_BLOCK = 1
_OFFSET_WORD = (24 << 8) | (31 << 16) | (62 << 24)


def _body(h_ref, dw_ref, dm_ref, mo_ref, o_ref):
    feat = jax.lax.broadcasted_iota(jnp.int32, (_WIDTH, 1), 0)
    s = ((feat >= 24).astype(jnp.int32)
         + (feat >= 31).astype(jnp.int32)
         + (feat >= 62).astype(jnp.int32)) << 3
    mask_c = jnp.int32(0xFF) << s
    target_c = feat << s

    w = (h_ref[...]
         + (dw_ref[...] << 8)
         + (dm_ref[...] << 16)
         + (mo_ref[...] << 24)
         + jnp.int32(_OFFSET_WORD))
    for h in range(_HIST_BLOCK):
        wrow = w[h]
        o_ref[h] = ((wrow & mask_c) == target_c).astype(jnp.float32)


@functools.partial(jax.jit, static_argnums=())
def kernel(hour, day_of_week, day_of_month, month):
    b, hist = hour.shape
    args = [x.astype(jnp.int32).T[:, None, :]
            for x in (hour, day_of_week, day_of_month, month)]

    in_spec = pl.BlockSpec((_HIST_BLOCK, 1, b), lambda i: (i, 0, 0))
    out_spec = pl.BlockSpec((_HIST_BLOCK, _WIDTH, b), lambda i: (i, 0, 0))
    out_t = pl.pallas_call(
        _body,
        grid=(hist // _HIST_BLOCK,),
        in_specs=[in_spec] * 4,
        out_specs=out_spec,
        out_shape=jax.ShapeDtypeStruct((hist, _WIDTH, b), jnp.float32),
    )(*args)
    return jnp.transpose(out_t, (2, 0, 1))


# speedup vs baseline: 1.3236x; 1.3236x over previous
import functools

import jax
import jax.numpy as jnp
from jax.experimental import pallas as pl

_HIST = 50
_WIDTH = 74
_BATCH_BLOCK = 256
_OFFSET_WORD = (24 << 8) | (31 << 16) | (62 << 24)


def _body(h_ref, dw_ref, dm_ref, mo_ref, o_ref):
    feat = jax.lax.broadcasted_iota(jnp.int32, (_WIDTH, 1), 0)
    s = ((feat >= 24).astype(jnp.int32)
         + (feat >= 31).astype(jnp.int32)
         + (feat >= 62).astype(jnp.int32)) << 3
    mask_c = jnp.int32(0xFF) << s
    target_c = feat << s

    w = (h_ref[...]
         + (dw_ref[...] << 8)
         + (dm_ref[...] << 16)
         + (mo_ref[...] << 24)
         + jnp.int32(_OFFSET_WORD))
    for h in range(_HIST):
        wrow = w[h:h + 1, :]
        o_ref[h] = ((wrow & mask_c) == target_c).astype(jnp.float32)


@functools.partial(jax.jit, static_argnums=())
def kernel(hour, day_of_week, day_of_month, month):
    b, hist = hour.shape
    args = [x.astype(jnp.int32).T for x in (hour, day_of_week, day_of_month, month)]

    in_spec = pl.BlockSpec((hist, _BATCH_BLOCK), lambda i: (0, i))
    out_spec = pl.BlockSpec((hist, _WIDTH, _BATCH_BLOCK), lambda i: (0, 0, i))
    out_t = pl.pallas_call(
        _body,
        grid=(b // _BATCH_BLOCK,),
        in_specs=[in_spec] * 4,
        out_specs=out_spec,
        out_shape=jax.ShapeDtypeStruct((hist, _WIDTH, b), jnp.float32),
    )(*args)
    return jnp.transpose(out_t, (2, 0, 1))


# speedup vs baseline: 1.4648x; 1.1067x over previous
import functools

import jax
import jax.numpy as jnp
from jax.experimental import pallas as pl

_HIST = 50
_WIDTH = 74
_BATCH_BLOCK = 1024
_OFFSET_WORD = (24 << 8) | (31 << 16) | (62 << 24)


def _body(h_ref, dw_ref, dm_ref, mo_ref, o_ref):
    feat = jax.lax.broadcasted_iota(jnp.int32, (_WIDTH, 1), 0)
    s = ((feat >= 24).astype(jnp.int32)
         + (feat >= 31).astype(jnp.int32)
         + (feat >= 62).astype(jnp.int32)) << 3
    mask_c = jnp.int32(0xFF) << s
    target_c = feat << s

    w = (h_ref[...]
         + (dw_ref[...] << 8)
         + (dm_ref[...] << 16)
         + (mo_ref[...] << 24)
         + jnp.int32(_OFFSET_WORD))
    for h in range(_HIST):
        wrow = w[h:h + 1, :]
        o_ref[h] = ((wrow & mask_c) == target_c).astype(jnp.float32)


@functools.partial(jax.jit, static_argnums=())
def kernel(hour, day_of_week, day_of_month, month):
    b, hist = hour.shape
    args = [x.astype(jnp.int32).T for x in (hour, day_of_week, day_of_month, month)]

    in_spec = pl.BlockSpec((hist, _BATCH_BLOCK), lambda i: (0, i))
    out_spec = pl.BlockSpec((hist, _WIDTH, _BATCH_BLOCK), lambda i: (0, 0, i))
    out_t = pl.pallas_call(
        _body,
        grid=(b // _BATCH_BLOCK,),
        in_specs=[in_spec] * 4,
        out_specs=out_spec,
        out_shape=jax.ShapeDtypeStruct((hist, _WIDTH, b), jnp.float32),
    )(*args)
    return jnp.transpose(out_t, (2, 0, 1))
